# baseline (speedup 1.0000x reference)
import functools

import jax
import jax.numpy as jnp
from jax import lax
from jax.experimental import pallas as pl
from jax.experimental.pallas import tpu as pltpu
from jax.experimental.pallas import tpu_sc as plsc

NC = 2
NS = 16
L = 16
NW = NC * NS


def kernel(x, pos_embedding, token_embedding):
    B, S = x.shape
    V, D = token_embedding.shape
    CH = S // NW
    PH = 16
    NPH = CH // PH
    RPP = B * PH

    xr = (x.astype(jnp.int32)
          .reshape(B, NW, NPH, PH)
          .transpose(1, 2, 0, 3)
          .reshape(NW, NPH, RPP))

    mesh = plsc.VectorSubcoreMesh(core_axis_name="c", subcore_axis_name="s")

    @functools.partial(
        pl.kernel,
        out_type=jax.ShapeDtypeStruct((B, S, D), jnp.float32),
        mesh=mesh,
        scratch_types=(
            [pltpu.VMEM((NPH, RPP), jnp.int32),
             pltpu.VMEM((CH, D), jnp.float32)]
            + [pltpu.VMEM((RPP, D), jnp.float32)] * NPH
            + [pltpu.SemaphoreType.DMA] * (NPH + 2)
        ),
    )
    def run(x_hbm, pos_hbm, tok_hbm, out_hbm, idx_v, pos_v, *rest):
        toks = list(rest[:NPH])
        gsems = list(rest[NPH:2 * NPH])
        ssem, psem = rest[2 * NPH:]

        wid = lax.axis_index("s") * NC + lax.axis_index("c")
        pbase = wid * CH

        pltpu.sync_copy(x_hbm.at[wid], idx_v)
        pcp = pltpu.async_copy(pos_hbm.at[pl.ds(pbase, CH)], pos_v, psem)

        gcp = [
            pltpu.async_copy(tok_hbm.at[idx_v.at[p]], toks[p], gsems[p])
            for p in range(NPH)
        ]
        pcp.wait()

        stcp = []
        for p in range(NPH):
            gcp[p].wait()
            tok_v = toks[p]

            def row_add(j, carry):
                pv = [pos_v[p * PH + j, pl.ds(k * L, L)] for k in range(D // L)]
                for b in range(B):
                    for k in range(D // L):
                        plsc.addupdate(
                            tok_v.at[b * PH + j, pl.ds(k * L, L)], pv[k])
                return carry

            lax.fori_loop(0, PH, row_add, 0)

            stcp.extend(
                pltpu.async_copy(
                    tok_v.at[pl.ds(b * PH, PH)],
                    out_hbm.at[b, pl.ds(pbase + p * PH, PH)], ssem)
                for b in range(B)
            )

        for h in stcp:
            h.wait()

    return run(xr, pos_embedding, token_embedding)


# speedup vs baseline: 1.0261x; 1.0261x over previous
import functools

import jax
import jax.numpy as jnp
from jax import lax
from jax.experimental import pallas as pl
from jax.experimental.pallas import tpu as pltpu
from jax.experimental.pallas import tpu_sc as plsc

NC = 2
NS = 16
L = 16
NW = NC * NS


def kernel(x, pos_embedding, token_embedding):
    B, S = x.shape
    V, D = token_embedding.shape
    CH = S // NW
    PH = 32
    NPH = CH // PH
    RPP = B * PH

    xr = (x.astype(jnp.int32)
          .reshape(B, NW, NPH, PH)
          .transpose(1, 2, 0, 3)
          .reshape(NW, NPH, RPP))

    mesh = plsc.VectorSubcoreMesh(core_axis_name="c", subcore_axis_name="s")

    @functools.partial(
        pl.kernel,
        out_type=jax.ShapeDtypeStruct((B, S, D), jnp.float32),
        mesh=mesh,
        scratch_types=(
            [pltpu.VMEM((NPH, RPP), jnp.int32),
             pltpu.VMEM((CH, D), jnp.float32)]
            + [pltpu.VMEM((RPP, D), jnp.float32)] * NPH
            + [pltpu.SemaphoreType.DMA] * (NPH + 2)
        ),
    )
    def run(x_hbm, pos_hbm, tok_hbm, out_hbm, idx_v, pos_v, *rest):
        toks = list(rest[:NPH])
        gsems = list(rest[NPH:2 * NPH])
        ssem, psem = rest[2 * NPH:]

        wid = lax.axis_index("s") * NC + lax.axis_index("c")
        pbase = wid * CH

        pltpu.sync_copy(x_hbm.at[wid], idx_v)
        pcp = pltpu.async_copy(pos_hbm.at[pl.ds(pbase, CH)], pos_v, psem)

        gcp = [
            pltpu.async_copy(tok_hbm.at[idx_v.at[p]], toks[p], gsems[p])
            for p in range(NPH)
        ]
        pcp.wait()

        stcp = []
        for p in range(NPH):
            gcp[p].wait()
            tok_v = toks[p]

            def row_add(j, carry):
                pv = [pos_v[p * PH + j, pl.ds(k * L, L)] for k in range(D // L)]
                for b in range(B):
                    for k in range(D // L):
                        plsc.addupdate(
                            tok_v.at[b * PH + j, pl.ds(k * L, L)], pv[k])
                return carry

            lax.fori_loop(0, PH, row_add, 0)

            stcp.extend(
                pltpu.async_copy(
                    tok_v.at[pl.ds(b * PH, PH)],
                    out_hbm.at[b, pl.ds(pbase + p * PH, PH)], ssem)
                for b in range(B)
            )

        for h in stcp:
            h.wait()

    return run(xr, pos_embedding, token_embedding)


# speedup vs baseline: 1.0320x; 1.0058x over previous
import functools

import jax
import jax.numpy as jnp
from jax import lax
from jax.experimental import pallas as pl
from jax.experimental.pallas import tpu as pltpu
from jax.experimental.pallas import tpu_sc as plsc

NC = 2
NS = 16
L = 16
NW = NC * NS


def kernel(x, pos_embedding, token_embedding):
    B, S = x.shape
    V, D = token_embedding.shape
    CH = S // NW
    PH = 32
    NPH = CH // PH
    RPP = B * PH

    xr = (x.astype(jnp.int32)
          .reshape(B, NW, NPH, PH)
          .transpose(1, 2, 0, 3)
          .reshape(NW, NPH, RPP))

    mesh = plsc.VectorSubcoreMesh(core_axis_name="c", subcore_axis_name="s")

    @functools.partial(
        pl.kernel,
        out_type=jax.ShapeDtypeStruct((B, S, D), jnp.float32),
        mesh=mesh,
        scratch_types=(
            [pltpu.VMEM((NPH, RPP), jnp.int32),
             pltpu.VMEM((CH, D), jnp.float32)]
            + [pltpu.VMEM((RPP, D), jnp.float32)] * NPH
            + [pltpu.SemaphoreType.DMA] * (NPH + 2)
        ),
    )
    def run(x_hbm, pos_hbm, tok_hbm, out_hbm, idx_v, pos_v, *rest):
        toks = list(rest[:NPH])
        gsems = list(rest[NPH:2 * NPH])
        ssem, psem = rest[2 * NPH:]

        wid = lax.axis_index("c") * NS + lax.axis_index("s")
        pbase = wid * CH

        pltpu.sync_copy(x_hbm.at[wid], idx_v)
        pcp = pltpu.async_copy(pos_hbm.at[pl.ds(pbase, CH)], pos_v, psem)

        gcp = [
            pltpu.async_copy(tok_hbm.at[idx_v.at[p]], toks[p], gsems[p])
            for p in range(NPH)
        ]
        pcp.wait()

        stcp = []
        for p in range(NPH):
            gcp[p].wait()
            tok_v = toks[p]

            def row_add(j, carry):
                pv = [pos_v[p * PH + j, pl.ds(k * L, L)] for k in range(D // L)]
                for b in range(B):
                    for k in range(D // L):
                        plsc.addupdate(
                            tok_v.at[b * PH + j, pl.ds(k * L, L)], pv[k])
                return carry

            lax.fori_loop(0, PH, row_add, 0)

            stcp.extend(
                pltpu.async_copy(
                    tok_v.at[pl.ds(b * PH, PH)],
                    out_hbm.at[b, pl.ds(pbase + p * PH, PH)], ssem)
                for b in range(B)
            )

        for h in stcp:
            h.wait()

    return run(xr, pos_embedding, token_embedding)
